# baseline (speedup 1.0000x reference)
import jax
import jax.numpy as jnp
from jax.experimental import pallas as pl
from jax.experimental.pallas import tpu as pltpu

BM = 200
NBUF = 4


def _gcn_body(inp_ref, w_ref, bias_ref, adj_hbm, out_ref,
              support_ref, buf_ref, sems):
    i = pl.program_id(0)
    nsteps = pl.num_programs(0)

    def start_copy(step):
        slot = jax.lax.rem(step, NBUF)
        pltpu.make_async_copy(
            adj_hbm.at[pl.ds(step * BM, BM), :],
            buf_ref.at[slot],
            sems.at[slot],
        ).start()

    @pl.when(i == 0)
    def _prologue():
        for s in range(NBUF - 1):
            start_copy(s)

    @pl.when(i + NBUF - 1 < nsteps)
    def _prefetch():
        start_copy(i + NBUF - 1)

    @pl.when(i == 0)
    def _compute_support():
        s = jnp.dot(
            inp_ref[...].astype(jnp.bfloat16),
            w_ref[...].astype(jnp.bfloat16),
            preferred_element_type=jnp.float32,
        )
        support_ref[...] = s.astype(jnp.bfloat16)

    slot = jax.lax.rem(i, NBUF)
    pltpu.make_async_copy(
        adj_hbm.at[pl.ds(i * BM, BM), :],
        buf_ref.at[slot],
        sems.at[slot],
    ).wait()
    acc = jnp.dot(
        buf_ref[slot].astype(jnp.bfloat16),
        support_ref[...],
        preferred_element_type=jnp.float32,
    )
    out_ref[...] = acc + bias_ref[...]


@jax.jit
def kernel(input, adj, weight, bias):
    n, f_in = input.shape
    f_out = weight.shape[1]
    bias2d = bias.reshape(1, f_out)
    grid = (n // BM,)
    out = pl.pallas_call(
        _gcn_body,
        grid=grid,
        in_specs=[
            pl.BlockSpec((n, f_in), lambda i: (0, 0)),
            pl.BlockSpec((f_in, f_out), lambda i: (0, 0)),
            pl.BlockSpec((1, f_out), lambda i: (0, 0)),
            pl.BlockSpec(memory_space=pl.ANY),
        ],
        out_specs=pl.BlockSpec((BM, f_out), lambda i: (i, 0)),
        out_shape=jax.ShapeDtypeStruct((n, f_out), jnp.float32),
        scratch_shapes=[
            pltpu.VMEM((n, f_out), jnp.bfloat16),
            pltpu.VMEM((NBUF, BM, n), jnp.float32),
            pltpu.SemaphoreType.DMA((NBUF,)),
        ],
        compiler_params=pltpu.CompilerParams(
            dimension_semantics=("arbitrary",),
            vmem_limit_bytes=100 * 1024 * 1024,
        ),
    )(input, weight, bias2d, adj)
    return out
